# Initial kernel scaffold; baseline (speedup 1.0000x reference)
import jax
import jax.numpy as jnp
from jax.experimental import pallas as pl


def kernel(x, batch_0, W_enc, b_enc, W_bb, b_bb):
    raise NotImplementedError("write your pallas kernel here")


# speedup vs baseline: 2.6610x; 2.6610x over previous
import functools

import jax
import jax.numpy as jnp
from jax import lax
from jax.experimental import pallas as pl
from jax.experimental.pallas import tpu as pltpu
from jax.experimental.pallas import tpu_sc as plsc

N = 100000
D = 128
G = 512

NC = 2
NS = 16
NW = NC * NS

ROWS_PER_STEP = 128
STEPS = 25
ROWS_PER_W = ROWS_PER_STEP * STEPS
N_PAD = NW * ROWS_PER_W

ACC_ROWS = G + 2 * NS
ZROWS = ACC_ROWS // NS

MM_BN = 800
MM_GRID = N_PAD // MM_BN
MM_LAST = N // MM_BN - 1


def _mm_body(x_ref, we_ref, be_ref, wb_ref, bb_ref, out_ref):
    h = jnp.dot(x_ref[...], we_ref[...], preferred_element_type=jnp.float32)
    h = jnp.maximum(h + be_ref[...], 0.0)
    out_ref[...] = jnp.dot(h, wb_ref[...],
                           preferred_element_type=jnp.float32) + bb_ref[...]


def _node_features(x, W_enc, b_enc, W_bb, b_bb):
    return pl.pallas_call(
        _mm_body,
        grid=(MM_GRID,),
        in_specs=[
            pl.BlockSpec((MM_BN, D), lambda i: (jnp.minimum(i, MM_LAST), 0)),
            pl.BlockSpec((D, D), lambda i: (0, 0)),
            pl.BlockSpec((1, D), lambda i: (0, 0)),
            pl.BlockSpec((D, D), lambda i: (0, 0)),
            pl.BlockSpec((1, D), lambda i: (0, 0)),
        ],
        out_specs=pl.BlockSpec((MM_BN, D), lambda i: (i, 0)),
        out_shape=jax.ShapeDtypeStruct((N_PAD, D), jnp.float32),
    )(x, W_enc, b_enc.reshape(1, D), W_bb, b_bb.reshape(1, D))


def _sc_body(nf_hbm, idx_hbm, part_hbm, cnt_hbm,
             idx_v, row_v, ones_v, zero_v, acc_sh, cnt_sh):
    cid = lax.axis_index("c")
    sid = lax.axis_index("s")
    wid = sid * NC + cid

    zeros16 = jnp.zeros((16,), jnp.float32)
    ones16 = jnp.ones((16,), jnp.float32)
    for r in range(ZROWS):
        for c in range(D // 16):
            zero_v[r, pl.ds(c * 16, 16)] = zeros16
    for r in range(ROWS_PER_STEP):
        for c in range(D // 16):
            ones_v[r, pl.ds(c * 16, 16)] = ones16

    pltpu.sync_copy(zero_v, acc_sh.at[pl.ds(sid * ZROWS, ZROWS)])
    pltpu.sync_copy(zero_v, cnt_sh.at[pl.ds(sid * ZROWS, ZROWS)])
    plsc.subcore_barrier()

    pltpu.sync_copy(idx_hbm.at[wid], idx_v)

    base = wid * ROWS_PER_W
    for j in range(STEPS):
        pltpu.sync_copy(nf_hbm.at[pl.ds(base + j * ROWS_PER_STEP,
                                        ROWS_PER_STEP)], row_v)
        pltpu.sync_copy(row_v, acc_sh.at[idx_v.at[j]], add=True)
        pltpu.sync_copy(ones_v, cnt_sh.at[idx_v.at[j]], add=True)

    plsc.subcore_barrier()

    rows_out = G // NS
    pltpu.sync_copy(acc_sh.at[pl.ds(sid * rows_out, rows_out)],
                    part_hbm.at[cid, pl.ds(sid * rows_out, rows_out)])
    pltpu.sync_copy(cnt_sh.at[pl.ds(sid * rows_out, rows_out)],
                    cnt_hbm.at[cid, pl.ds(sid * rows_out, rows_out)])


def _segment_partials(nf, idx):
    mesh = plsc.VectorSubcoreMesh(core_axis_name="c", subcore_axis_name="s",
                                  num_cores=NC, num_subcores=NS)
    k = functools.partial(
        pl.kernel,
        out_type=[jax.ShapeDtypeStruct((NC, G, D), jnp.float32),
                  jax.ShapeDtypeStruct((NC, G, D), jnp.float32)],
        mesh=mesh,
        scratch_types=[
            pltpu.VMEM((STEPS, ROWS_PER_STEP), jnp.int32),
            pltpu.VMEM((ROWS_PER_STEP, D), jnp.float32),
            pltpu.VMEM((ROWS_PER_STEP, D), jnp.float32),
            pltpu.VMEM((ZROWS, D), jnp.float32),
            pltpu.VMEM_SHARED((ACC_ROWS, D), jnp.float32),
            pltpu.VMEM_SHARED((ACC_ROWS, D), jnp.float32),
        ],
    )(_sc_body)
    return k(nf, idx)


def _comb_body(p_ref, c_ref, out_ref):
    s = p_ref[0] + p_ref[1]
    cnt = c_ref[0, :, 0:1] + c_ref[1, :, 0:1]
    out_ref[...] = s / jnp.maximum(cnt, 1.0)


def _combine(part, cnt):
    return pl.pallas_call(
        _comb_body,
        out_shape=jax.ShapeDtypeStruct((G, D), jnp.float32),
    )(part, cnt)


def kernel(x, batch_0, W_enc, b_enc, W_bb, b_bb):
    nf = _node_features(x, W_enc, b_enc, W_bb, b_bb)
    idx = jnp.concatenate(
        [batch_0, jnp.full((N_PAD - N,), G, jnp.int32)]).reshape(
            NW, STEPS, ROWS_PER_STEP)
    part, cnt = _segment_partials(nf, idx)
    return _combine(part, cnt)


# speedup vs baseline: 3.0800x; 1.1575x over previous
import functools

import jax
import jax.numpy as jnp
from jax import lax
from jax.experimental import pallas as pl
from jax.experimental.pallas import tpu as pltpu
from jax.experimental.pallas import tpu_sc as plsc

N = 100000
D = 128
G = 512

NC = 2
NS = 16
NW = NC * NS

ROWS_PER_STEP = 128
STEPS = 25
ROWS_PER_W = ROWS_PER_STEP * STEPS
N_PAD = NW * ROWS_PER_W

ACC_ROWS = G + 2 * NS
ZROWS = ACC_ROWS // NS

MM_BN = 800
MM_GRID = N_PAD // MM_BN
MM_LAST = N // MM_BN - 1


def _mm_body(x_ref, we_ref, be_ref, wb_ref, bb_ref, ids_ref, out_ref, cnt_ref):
    i = pl.program_id(0)
    h = jnp.dot(x_ref[...], we_ref[...], preferred_element_type=jnp.float32)
    h = jnp.maximum(h + be_ref[...], 0.0)
    out_ref[...] = jnp.dot(h, wb_ref[...],
                           preferred_element_type=jnp.float32) + bb_ref[...]

    ids = ids_ref[0, 0, :]
    seg = lax.broadcasted_iota(jnp.int32, (G, MM_BN), 0)
    oh = jnp.where(seg == ids[None, :], 1.0, 0.0)
    blk_cnt = jnp.sum(oh, axis=1, keepdims=True)

    @pl.when(i == 0)
    def _():
        cnt_ref[...] = jnp.zeros_like(cnt_ref)

    cnt_ref[...] += blk_cnt


def _node_features_and_counts(x, W_enc, b_enc, W_bb, b_bb, ids):
    return pl.pallas_call(
        _mm_body,
        grid=(MM_GRID,),
        in_specs=[
            pl.BlockSpec((MM_BN, D), lambda i: (jnp.minimum(i, MM_LAST), 0)),
            pl.BlockSpec((D, D), lambda i: (0, 0)),
            pl.BlockSpec((1, D), lambda i: (0, 0)),
            pl.BlockSpec((D, D), lambda i: (0, 0)),
            pl.BlockSpec((1, D), lambda i: (0, 0)),
            pl.BlockSpec((1, 1, MM_BN), lambda i: (i, 0, 0)),
        ],
        out_specs=[
            pl.BlockSpec((MM_BN, D), lambda i: (i, 0)),
            pl.BlockSpec((G, 1), lambda i: (0, 0)),
        ],
        out_shape=[
            jax.ShapeDtypeStruct((N_PAD, D), jnp.float32),
            jax.ShapeDtypeStruct((G, 1), jnp.float32),
        ],
    )(x, W_enc, b_enc.reshape(1, D), W_bb, b_bb.reshape(1, D), ids)


def _sc_body(nf_hbm, idx_hbm, part_hbm,
             idx_v, row_v0, row_v1, zero_v, sem0, sem1, acc_sh):
    cid = lax.axis_index("c")
    sid = lax.axis_index("s")
    wid = sid * NC + cid

    zeros16 = jnp.zeros((16,), jnp.float32)
    for r in range(ZROWS):
        for c in range(D // 16):
            zero_v[r, pl.ds(c * 16, 16)] = zeros16
    pltpu.sync_copy(zero_v, acc_sh.at[pl.ds(sid * ZROWS, ZROWS)])
    plsc.subcore_barrier()

    pltpu.sync_copy(idx_hbm.at[wid], idx_v)

    base = wid * ROWS_PER_W
    bufs = (row_v0, row_v1)
    sems = (sem0, sem1)
    handles = [None, None]
    handles[0] = pltpu.async_copy(
        nf_hbm.at[pl.ds(base, ROWS_PER_STEP)], row_v0, sem0)
    for j in range(STEPS):
        b = bufs[j % 2]
        handles[j % 2].wait()
        if j + 1 < STEPS:
            handles[(j + 1) % 2] = pltpu.async_copy(
                nf_hbm.at[pl.ds(base + (j + 1) * ROWS_PER_STEP,
                                ROWS_PER_STEP)],
                bufs[(j + 1) % 2], sems[(j + 1) % 2])
        pltpu.sync_copy(b, acc_sh.at[idx_v.at[j]], add=True)

    plsc.subcore_barrier()

    rows_out = G // NS
    pltpu.sync_copy(acc_sh.at[pl.ds(sid * rows_out, rows_out)],
                    part_hbm.at[cid, pl.ds(sid * rows_out, rows_out)])


def _segment_partials(nf, idx):
    mesh = plsc.VectorSubcoreMesh(core_axis_name="c", subcore_axis_name="s",
                                  num_cores=NC, num_subcores=NS)
    k = functools.partial(
        pl.kernel,
        out_type=jax.ShapeDtypeStruct((NC, G, D), jnp.float32),
        mesh=mesh,
        scratch_types=[
            pltpu.VMEM((STEPS, ROWS_PER_STEP), jnp.int32),
            pltpu.VMEM((ROWS_PER_STEP, D), jnp.float32),
            pltpu.VMEM((ROWS_PER_STEP, D), jnp.float32),
            pltpu.VMEM((ZROWS, D), jnp.float32),
            pltpu.SemaphoreType.DMA,
            pltpu.SemaphoreType.DMA,
            pltpu.VMEM_SHARED((ACC_ROWS, D), jnp.float32),
        ],
    )(_sc_body)
    return k(nf, idx)


def _comb_body(p_ref, c_ref, out_ref):
    s = p_ref[0] + p_ref[1]
    out_ref[...] = s / jnp.maximum(c_ref[...], 1.0)


def _combine(part, cnt):
    return pl.pallas_call(
        _comb_body,
        out_shape=jax.ShapeDtypeStruct((G, D), jnp.float32),
    )(part, cnt)


def kernel(x, batch_0, W_enc, b_enc, W_bb, b_bb):
    idx_pad = jnp.concatenate(
        [batch_0, jnp.full((N_PAD - N,), G, jnp.int32)])
    ids_tc = idx_pad.reshape(MM_GRID, 1, MM_BN)
    idx_sc = idx_pad.reshape(NW, STEPS, ROWS_PER_STEP)
    nf, cnt = _node_features_and_counts(x, W_enc, b_enc, W_bb, b_bb, ids_tc)
    part = _segment_partials(nf, idx_sc)
    return _combine(part, cnt)
